# Initial kernel scaffold; baseline (speedup 1.0000x reference)
import jax
import jax.numpy as jnp
from jax import lax
from jax.experimental import pallas as pl
from jax.experimental.pallas import tpu as pltpu
from jax.experimental.pallas import tpu_sc as plsc

N = 10000
E = 320000
D = 128
DE = 16
H = 128
FH = 128
B = 512

NC = 2
NS = 16
NW = NC * NS
K = 128
NBUF = 4
RB = 512
NP = 10240
CH = (-(-E // (NW * K)) + NBUF - 1) // NBUF * NBUF
EP = NW * CH * K
SR = NP // NS

_PREC = lax.Precision.HIGHEST


def _dot(a, b):
    return jnp.dot(a, b, preferred_element_type=jnp.float32, precision=_PREC)


def _make_sc_round(with_ea: bool):
    mesh = plsc.VectorSubcoreMesh(core_axis_name="c", subcore_axis_name="s")

    out_type = [jax.ShapeDtypeStruct((NC, NP, H), jnp.float32)]
    scratch = [
        pltpu.VMEM((CH, K), jnp.int32),
        pltpu.VMEM((CH, K), jnp.int32),
        pltpu.VMEM((NBUF, K, H), jnp.float32),
        pltpu.VMEM_SHARED((NP, H), jnp.float32),
    ] + [pltpu.SemaphoreType.DMA] * NBUF
    if with_ea:
        out_type.append(jax.ShapeDtypeStruct((NC, NP, DE), jnp.float32))
        scratch += [
            pltpu.VMEM((NBUF, K, DE), jnp.float32),
            pltpu.VMEM_SHARED((NP, DE), jnp.float32),
        ] + [pltpu.SemaphoreType.DMA] * NBUF

    def body(*refs):
        if with_ea:
            (src_hbm, dst_hbm, h_hbm, zh_hbm, ze_hbm, ea_hbm,
             out_hbm, eaout_hbm,
             src_v, dst_v, rows_v, acc_sh) = refs[:12]
            gsems = refs[12:12 + NBUF]
            eav, eacc_sh = refs[12 + NBUF:14 + NBUF]
            esems = refs[14 + NBUF:]
        else:
            (src_hbm, dst_hbm, h_hbm, zh_hbm,
             out_hbm, src_v, dst_v, rows_v, acc_sh) = refs[:9]
            gsems = refs[9:]

        c = lax.axis_index("c")
        s = lax.axis_index("s")
        w = c * NS + s

        pltpu.sync_copy(src_hbm.at[w], src_v)
        pltpu.sync_copy(dst_hbm.at[w], dst_v)
        pltpu.sync_copy(zh_hbm, acc_sh.at[pl.ds(s * SR, SR)])
        if with_ea:
            pltpu.sync_copy(ze_hbm, eacc_sh.at[pl.ds(s * SR, SR)])
        plsc.subcore_barrier()

        for b in range(NBUF):
            pltpu.async_copy(h_hbm.at[src_v.at[b]], rows_v.at[b], gsems[b])
            if with_ea:
                pltpu.async_copy(ea_hbm.at[w, pl.ds(b * K, K)], eav.at[b],
                                 esems[b])

        @pl.loop(0, CH, step=NBUF)
        def _(j0):
            for b in range(NBUF):
                j = j0 + b
                pltpu.make_async_copy(
                    h_hbm.at[src_v.at[j]], rows_v.at[b], gsems[b]).wait()
                pltpu.sync_copy(rows_v.at[b], acc_sh.at[dst_v.at[j]],
                                add=True)
                if with_ea:
                    pltpu.make_async_copy(
                        ea_hbm.at[w, pl.ds(j * K, K)], eav.at[b],
                        esems[b]).wait()
                    pltpu.sync_copy(eav.at[b], eacc_sh.at[dst_v.at[j]],
                                    add=True)

                @pl.when(j + NBUF < CH)
                def _():
                    pltpu.async_copy(h_hbm.at[src_v.at[j + NBUF]],
                                     rows_v.at[b], gsems[b])
                    if with_ea:
                        pltpu.async_copy(
                            ea_hbm.at[w, pl.ds((j + NBUF) * K, K)],
                            eav.at[b], esems[b])

        plsc.subcore_barrier()
        row0 = s * SR
        pltpu.sync_copy(acc_sh.at[pl.ds(row0, SR)],
                        out_hbm.at[c, pl.ds(row0, SR)])
        if with_ea:
            pltpu.sync_copy(eacc_sh.at[pl.ds(row0, SR)],
                            eaout_hbm.at[c, pl.ds(row0, SR)])

    return pl.kernel(body, out_type=tuple(out_type), mesh=mesh,
                     scratch_types=scratch)


_sc_round_ea = _make_sc_round(True)
_sc_round = _make_sc_round(False)


def _tc1_body(x_ref, wi_ref, o_ref):
    o_ref[...] = jnp.maximum(_dot(x_ref[...], wi_ref[...]), 0.0)


def _tc1(x_p, W_i):
    return pl.pallas_call(
        _tc1_body,
        grid=(NP // RB,),
        in_specs=[
            pl.BlockSpec((RB, D), lambda i: (i, 0)),
            pl.BlockSpec((D, H), lambda i: (0, 0)),
        ],
        out_specs=pl.BlockSpec((RB, H), lambda i: (i, 0)),
        out_shape=jax.ShapeDtypeStruct((NP, H), jnp.float32),
    )(x_p, W_i)


def _round_update(h0, a_ref, ea_ref, wht_ref, whb_ref):
    a = a_ref[0] + a_ref[1]
    ea = ea_ref[0] + ea_ref[1]
    return jnp.maximum(h0 + _dot(a, wht_ref[...]) + _dot(ea, whb_ref[...]),
                       0.0)


def _tc2_body(h0_ref, a_ref, ea_ref, wht_ref, whb_ref, o_ref):
    o_ref[...] = _round_update(h0_ref[...], a_ref, ea_ref, wht_ref, whb_ref)


def _tc2(h0, A, EA, wht, whb):
    return pl.pallas_call(
        _tc2_body,
        grid=(NP // RB,),
        in_specs=[
            pl.BlockSpec((RB, H), lambda i: (i, 0)),
            pl.BlockSpec((NC, RB, H), lambda i: (0, i, 0)),
            pl.BlockSpec((NC, RB, DE), lambda i: (0, i, 0)),
            pl.BlockSpec((H, H), lambda i: (0, 0)),
            pl.BlockSpec((DE, H), lambda i: (0, 0)),
        ],
        out_specs=pl.BlockSpec((RB, H), lambda i: (i, 0)),
        out_shape=jax.ShapeDtypeStruct((NP, H), jnp.float32),
    )(h0, A, EA, wht, whb)


def _tc3_body(h0_ref, a_ref, ea_ref, x_ref, mol_ref, wht_ref, whb_ref,
              woa_ref, wob_ref, w1_ref, b1_ref, w2_ref, b2_ref,
              o_ref, acc, cnt):
    i = pl.program_id(0)

    @pl.when(i == 0)
    def _():
        acc[...] = jnp.zeros_like(acc)
        cnt[...] = jnp.zeros_like(cnt)

    h2 = _round_update(h0_ref[...], a_ref, ea_ref, wht_ref, whb_ref)
    atom = jnp.maximum(_dot(x_ref[...], woa_ref[...]) +
                       _dot(h2, wob_ref[...]), 0.0)
    mol_row = mol_ref[0]
    mol_ids = lax.broadcasted_iota(jnp.int32, (B, 1), 0)
    oht = (mol_ids == mol_row).astype(jnp.float32)
    acc[...] += _dot(oht, atom)
    cnt[...] += jnp.sum(oht, axis=1, keepdims=True)

    @pl.when(i == NP // RB - 1)
    def _():
        mol_vec = acc[...] / jnp.maximum(cnt[...], 1.0)
        hid = jnp.maximum(_dot(mol_vec, w1_ref[...]) + b1_ref[...], 0.0)
        o_ref[...] = _dot(hid, w2_ref[...]) + b2_ref[...]


def _tc3(h0, A, EA, x_p, mol3, wht, whb, woa, wob, W1, b1r, W2, b2r):
    return pl.pallas_call(
        _tc3_body,
        grid=(NP // RB,),
        in_specs=[
            pl.BlockSpec((RB, H), lambda i: (i, 0)),
            pl.BlockSpec((NC, RB, H), lambda i: (0, i, 0)),
            pl.BlockSpec((NC, RB, DE), lambda i: (0, i, 0)),
            pl.BlockSpec((RB, D), lambda i: (i, 0)),
            pl.BlockSpec((1, 1, RB), lambda i: (i, 0, 0)),
            pl.BlockSpec((H, H), lambda i: (0, 0)),
            pl.BlockSpec((DE, H), lambda i: (0, 0)),
            pl.BlockSpec((D, H), lambda i: (0, 0)),
            pl.BlockSpec((H, H), lambda i: (0, 0)),
            pl.BlockSpec((H, FH), lambda i: (0, 0)),
            pl.BlockSpec((1, FH), lambda i: (0, 0)),
            pl.BlockSpec((FH, 1), lambda i: (0, 0)),
            pl.BlockSpec((1, 1), lambda i: (0, 0)),
        ],
        out_specs=pl.BlockSpec((B, 1), lambda i: (0, 0)),
        out_shape=jax.ShapeDtypeStruct((B, 1), jnp.float32),
        scratch_shapes=[
            pltpu.VMEM((B, H), jnp.float32),
            pltpu.VMEM((B, 1), jnp.float32),
        ],
    )(h0, A, EA, x_p, mol3, wht, whb, woa, wob, W1, b1r, W2, b2r)


def kernel(x, edge_index, edge_attr, mol_batch, W_i, W_h, W_o, W1, b1, W2,
           b2):
    f32 = jnp.float32
    npad_e = EP - E
    pad_idx = N + (jnp.arange(npad_e, dtype=jnp.int32) % (NP - N))
    src_p = jnp.concatenate([edge_index[0], pad_idx]).reshape(NW, CH, K)
    dst_p = jnp.concatenate([edge_index[1], pad_idx]).reshape(NW, CH, K)
    ea_p = jnp.concatenate(
        [edge_attr, jnp.zeros((npad_e, DE), f32)]).reshape(NW, CH * K, DE)
    x_p = jnp.concatenate([x, jnp.zeros((NP - N, D), f32)])
    mol3 = jnp.concatenate(
        [mol_batch, jnp.full((NP - N,), B, jnp.int32)]).reshape(
            NP // RB, 1, RB)
    zh = jnp.zeros((SR, H), f32)
    ze = jnp.zeros((SR, DE), f32)
    wht, whb = W_h[:H], W_h[H:]
    woa, wob = W_o[:D], W_o[D:]
    b1r = b1.reshape(1, FH)
    b2r = b2.reshape(1, 1)

    h0 = _tc1(x_p, W_i)
    A1, EA = _sc_round_ea(src_p, dst_p, h0, zh, ze, ea_p)
    h1 = _tc2(h0, A1, EA, wht, whb)
    (A2,) = _sc_round(src_p, dst_p, h1, zh)
    return _tc3(h0, A2, EA, x_p, mol3, wht, whb, woa, wob, W1, b1r, W2, b2r)


# speedup vs baseline: 7.7081x; 7.7081x over previous
import functools

import jax
import jax.numpy as jnp
from jax import lax
from jax.experimental import pallas as pl
from jax.experimental.pallas import tpu as pltpu
from jax.experimental.pallas import tpu_sc as plsc

N = 10000
E = 320000
D = 128
DE = 16
H = 128
FH = 128
B = 512

NC = 2
NS = 16
NW = NC * NS
K = 128
NBUF = 4
RB = 512
NP = 10240
CH = (-(-E // (NW * K)) + NBUF - 1) // NBUF * NBUF
EP = NW * CH * K
CH2 = EP // (NS * K)
SR = NP // NS
HH = H // NC

_PREC = lax.Precision.HIGHEST


def _dot(a, b):
    return jnp.dot(a, b, preferred_element_type=jnp.float32, precision=_PREC)


def _mesh():
    return plsc.VectorSubcoreMesh(core_axis_name="c", subcore_axis_name="s",
                                  num_cores=NC, num_subcores=NS)


@functools.cache
def _make_sc_round():
    scratch = [
        pltpu.VMEM((CH2, K), jnp.int32),
        pltpu.VMEM((CH2, K), jnp.int32),
        pltpu.VMEM((NBUF, K, HH), jnp.float32),
        pltpu.VMEM_SHARED((NP, HH), jnp.float32),
    ] + [pltpu.SemaphoreType.DMA] * NBUF

    def body(src_hbm, dst_hbm, h_hbm, zh_hbm, out_hbm,
             src_v, dst_v, rows_v, acc_sh, *gsems):
        c = lax.axis_index("c")
        s = lax.axis_index("s")

        pltpu.sync_copy(src_hbm.at[c, s], src_v)
        pltpu.sync_copy(dst_hbm.at[s], dst_v)
        pltpu.sync_copy(zh_hbm, acc_sh.at[pl.ds(s * SR, SR)])
        plsc.subcore_barrier()

        for b in range(NBUF):
            pltpu.async_copy(h_hbm.at[src_v.at[b]], rows_v.at[b], gsems[b])

        @pl.loop(0, CH2, step=NBUF)
        def _(j0):
            for b in range(NBUF):
                j = j0 + b
                pltpu.make_async_copy(
                    h_hbm.at[src_v.at[j]], rows_v.at[b], gsems[b]).wait()
                pltpu.sync_copy(rows_v.at[b], acc_sh.at[dst_v.at[j]],
                                add=True)

                @pl.when(j + NBUF < CH2)
                def _():
                    pltpu.async_copy(h_hbm.at[src_v.at[j + NBUF]],
                                     rows_v.at[b], gsems[b])

        plsc.subcore_barrier()
        row0 = s * SR
        pltpu.sync_copy(acc_sh.at[pl.ds(row0, SR)],
                        out_hbm.at[c, pl.ds(row0, SR)])

    return pl.kernel(body,
                     out_type=jax.ShapeDtypeStruct((NC, NP, HH), jnp.float32),
                     mesh=_mesh(), scratch_types=scratch,
                     compiler_params=pltpu.CompilerParams(
                         use_tc_tiling_on_sc=False))


@functools.cache
def _make_sc_ea():
    scratch = [
        pltpu.VMEM((CH, K), jnp.int32),
        pltpu.VMEM((NBUF, K, DE), jnp.float32),
        pltpu.VMEM_SHARED((NP, DE), jnp.float32),
    ] + [pltpu.SemaphoreType.DMA] * NBUF

    def body(dst_hbm, ea_hbm, ze_hbm, out_hbm,
             dst_v, eav, acc_sh, *esems):
        c = lax.axis_index("c")
        s = lax.axis_index("s")
        w = c * NS + s

        pltpu.sync_copy(dst_hbm.at[w], dst_v)
        pltpu.sync_copy(ze_hbm, acc_sh.at[pl.ds(s * SR, SR)])
        plsc.subcore_barrier()

        for b in range(NBUF):
            pltpu.async_copy(ea_hbm.at[w, pl.ds(b * K, K)], eav.at[b],
                             esems[b])

        @pl.loop(0, CH, step=NBUF)
        def _(j0):
            for b in range(NBUF):
                j = j0 + b
                pltpu.make_async_copy(
                    ea_hbm.at[w, pl.ds(j * K, K)], eav.at[b],
                    esems[b]).wait()
                pltpu.sync_copy(eav.at[b], acc_sh.at[dst_v.at[j]], add=True)

                @pl.when(j + NBUF < CH)
                def _():
                    pltpu.async_copy(ea_hbm.at[w, pl.ds((j + NBUF) * K, K)],
                                     eav.at[b], esems[b])

        plsc.subcore_barrier()
        row0 = s * SR
        pltpu.sync_copy(acc_sh.at[pl.ds(row0, SR)],
                        out_hbm.at[c, pl.ds(row0, SR)])

    return pl.kernel(body,
                     out_type=jax.ShapeDtypeStruct((NC, NP, DE), jnp.float32),
                     mesh=_mesh(), scratch_types=scratch,
                     compiler_params=pltpu.CompilerParams(
                         use_tc_tiling_on_sc=False))


def _sc_round(src_sp, dst_sp, h_split, zh):
    h_cat = h_split.reshape(NC * NP, HH)
    return _make_sc_round()(src_sp, dst_sp, h_cat, zh)


def _sc_ea(dst_p, ea_p, ze):
    return _make_sc_ea()(dst_p, ea_p, ze)


def _split_out(o_ref, vals):
    o_ref[0] = vals[:, :HH]
    o_ref[1] = vals[:, HH:]


def _cat(ref):
    return jnp.concatenate([ref[0], ref[1]], axis=-1)


def _tc1_body(x_ref, wi_ref, o_ref):
    _split_out(o_ref, jnp.maximum(_dot(x_ref[...], wi_ref[...]), 0.0))


def _tc1(x_p, W_i):
    return pl.pallas_call(
        _tc1_body,
        grid=(NP // RB,),
        in_specs=[
            pl.BlockSpec((RB, D), lambda i: (i, 0)),
            pl.BlockSpec((D, H), lambda i: (0, 0)),
        ],
        out_specs=pl.BlockSpec((NC, RB, HH), lambda i: (0, i, 0)),
        out_shape=jax.ShapeDtypeStruct((NC, NP, HH), jnp.float32),
    )(x_p, W_i)


def _round_update(h0_ref, a_ref, ea_ref, wht_ref, whb_ref):
    a = _cat(a_ref)
    ea = ea_ref[0] + ea_ref[1]
    return jnp.maximum(
        _cat(h0_ref) + _dot(a, wht_ref[...]) + _dot(ea, whb_ref[...]), 0.0)


def _tc2_body(h0_ref, a_ref, ea_ref, wht_ref, whb_ref, o_ref):
    _split_out(o_ref, _round_update(h0_ref, a_ref, ea_ref, wht_ref, whb_ref))


def _tc2(h0, A, EA, wht, whb):
    return pl.pallas_call(
        _tc2_body,
        grid=(NP // RB,),
        in_specs=[
            pl.BlockSpec((NC, RB, HH), lambda i: (0, i, 0)),
            pl.BlockSpec((NC, RB, HH), lambda i: (0, i, 0)),
            pl.BlockSpec((NC, RB, DE), lambda i: (0, i, 0)),
            pl.BlockSpec((H, H), lambda i: (0, 0)),
            pl.BlockSpec((DE, H), lambda i: (0, 0)),
        ],
        out_specs=pl.BlockSpec((NC, RB, HH), lambda i: (0, i, 0)),
        out_shape=jax.ShapeDtypeStruct((NC, NP, HH), jnp.float32),
    )(h0, A, EA, wht, whb)


def _tc3_body(h0_ref, a_ref, ea_ref, x_ref, mol_ref, wht_ref, whb_ref,
              woa_ref, wob_ref, w1_ref, b1_ref, w2_ref, b2_ref,
              o_ref, acc, cnt):
    i = pl.program_id(0)

    @pl.when(i == 0)
    def _():
        acc[...] = jnp.zeros_like(acc)
        cnt[...] = jnp.zeros_like(cnt)

    h2 = _round_update(h0_ref, a_ref, ea_ref, wht_ref, whb_ref)
    atom = jnp.maximum(_dot(x_ref[...], woa_ref[...]) +
                       _dot(h2, wob_ref[...]), 0.0)
    mol_row = mol_ref[0]
    mol_ids = lax.broadcasted_iota(jnp.int32, (B, 1), 0)
    oht = (mol_ids == mol_row).astype(jnp.float32)
    acc[...] += _dot(oht, atom)
    cnt[...] += jnp.sum(oht, axis=1, keepdims=True)

    @pl.when(i == NP // RB - 1)
    def _():
        mol_vec = acc[...] / jnp.maximum(cnt[...], 1.0)
        hid = jnp.maximum(_dot(mol_vec, w1_ref[...]) + b1_ref[...], 0.0)
        o_ref[...] = _dot(hid, w2_ref[...]) + b2_ref[...]


def _tc3(h0, A, EA, x_p, mol3, wht, whb, woa, wob, W1, b1r, W2, b2r):
    return pl.pallas_call(
        _tc3_body,
        grid=(NP // RB,),
        in_specs=[
            pl.BlockSpec((NC, RB, HH), lambda i: (0, i, 0)),
            pl.BlockSpec((NC, RB, HH), lambda i: (0, i, 0)),
            pl.BlockSpec((NC, RB, DE), lambda i: (0, i, 0)),
            pl.BlockSpec((RB, D), lambda i: (i, 0)),
            pl.BlockSpec((1, 1, RB), lambda i: (i, 0, 0)),
            pl.BlockSpec((H, H), lambda i: (0, 0)),
            pl.BlockSpec((DE, H), lambda i: (0, 0)),
            pl.BlockSpec((D, H), lambda i: (0, 0)),
            pl.BlockSpec((H, H), lambda i: (0, 0)),
            pl.BlockSpec((H, FH), lambda i: (0, 0)),
            pl.BlockSpec((1, FH), lambda i: (0, 0)),
            pl.BlockSpec((FH, 1), lambda i: (0, 0)),
            pl.BlockSpec((1, 1), lambda i: (0, 0)),
        ],
        out_specs=pl.BlockSpec((B, 1), lambda i: (0, 0)),
        out_shape=jax.ShapeDtypeStruct((B, 1), jnp.float32),
        scratch_shapes=[
            pltpu.VMEM((B, H), jnp.float32),
            pltpu.VMEM((B, 1), jnp.float32),
        ],
    )(h0, A, EA, x_p, mol3, wht, whb, woa, wob, W1, b1r, W2, b2r)


def kernel(x, edge_index, edge_attr, mol_batch, W_i, W_h, W_o, W1, b1, W2,
           b2):
    f32 = jnp.float32
    npad_e = EP - E
    pad_idx = N + (jnp.arange(npad_e, dtype=jnp.int32) % (NP - N))
    src_f = jnp.concatenate([edge_index[0], pad_idx])
    dst_f = jnp.concatenate([edge_index[1], pad_idx])
    src_sp = jnp.stack(
        [src_f + c * NP for c in range(NC)]).reshape(NC, NS, CH2, K)
    dst_sp = dst_f.reshape(NS, CH2, K)
    dst_w = dst_f.reshape(NW, CH, K)
    ea_p = jnp.concatenate(
        [edge_attr, jnp.zeros((npad_e, DE), f32)]).reshape(NW, CH * K, DE)
    x_p = jnp.concatenate([x, jnp.zeros((NP - N, D), f32)])
    mol3 = jnp.concatenate(
        [mol_batch, jnp.full((NP - N,), B, jnp.int32)]).reshape(
            NP // RB, 1, RB)
    zh = jnp.zeros((SR, HH), f32)
    ze = jnp.zeros((SR, DE), f32)
    wht, whb = W_h[:H], W_h[H:]
    woa, wob = W_o[:D], W_o[D:]
    b1r = b1.reshape(1, FH)
    b2r = b2.reshape(1, 1)

    h0 = _tc1(x_p, W_i)
    EA = _sc_ea(dst_w, ea_p, ze)
    A1 = _sc_round(src_sp, dst_sp, h0, zh)
    h1 = _tc2(h0, A1, EA, wht, whb)
    A2 = _sc_round(src_sp, dst_sp, h1, zh)
    return _tc3(h0, A2, EA, x_p, mol3, wht, whb, woa, wob, W1, b1r, W2, b2r)


# speedup vs baseline: 10.0106x; 1.2987x over previous
import functools

import jax
import jax.numpy as jnp
from jax import lax
from jax.experimental import pallas as pl
from jax.experimental.pallas import tpu as pltpu
from jax.experimental.pallas import tpu_sc as plsc

N = 10000
E = 320000
D = 128
DE = 16
H = 128
FH = 128
B = 512

NC = 2
NS = 16
NW = NC * NS
K = 128
NBUF = 4
RB = 512
NP = 10240
CH = (-(-E // (NW * K)) + NBUF - 1) // NBUF * NBUF
EP = NW * CH * K
CH2 = EP // (NS * K)
SR = NP // NS
HH = H // NC
EW = E // NW
CHE = (-(-EW // K) + NBUF - 1) // NBUF * NBUF

_PREC = lax.Precision.HIGHEST


def _dot(a, b):
    return jnp.dot(a, b, preferred_element_type=jnp.float32, precision=_PREC)


def _mesh():
    return plsc.VectorSubcoreMesh(core_axis_name="c", subcore_axis_name="s",
                                  num_cores=NC, num_subcores=NS)


@functools.cache
def _make_sc_round():
    scratch = [
        pltpu.VMEM((CH2, K), jnp.int32),
        pltpu.VMEM((CH2, K), jnp.int32),
        pltpu.VMEM((NBUF, K, HH), jnp.float32),
        pltpu.VMEM_SHARED((NP, HH), jnp.float32),
    ] + [pltpu.SemaphoreType.DMA] * NBUF

    def body(src_hbm, dst_hbm, h_hbm, zh_hbm, out_hbm,
             src_v, dst_v, rows_v, acc_sh, *gsems):
        c = lax.axis_index("c")
        s = lax.axis_index("s")

        pltpu.sync_copy(src_hbm.at[c, s], src_v)
        pltpu.sync_copy(dst_hbm.at[s], dst_v)
        pltpu.sync_copy(zh_hbm, acc_sh.at[pl.ds(s * SR, SR)])
        plsc.subcore_barrier()

        for b in range(NBUF):
            pltpu.async_copy(h_hbm.at[src_v.at[b]], rows_v.at[b], gsems[b])

        @pl.loop(0, CH2, step=NBUF)
        def _(j0):
            for b in range(NBUF):
                j = j0 + b
                pltpu.make_async_copy(
                    h_hbm.at[src_v.at[j]], rows_v.at[b], gsems[b]).wait()
                pltpu.sync_copy(rows_v.at[b], acc_sh.at[dst_v.at[j]],
                                add=True)

                @pl.when(j + NBUF < CH2)
                def _():
                    pltpu.async_copy(h_hbm.at[src_v.at[j + NBUF]],
                                     rows_v.at[b], gsems[b])

        plsc.subcore_barrier()
        row0 = s * SR
        pltpu.sync_copy(acc_sh.at[pl.ds(row0, SR)],
                        out_hbm.at[c, pl.ds(row0, SR)])

    return pl.kernel(body,
                     out_type=jax.ShapeDtypeStruct((NC, NP, HH), jnp.float32),
                     mesh=_mesh(), scratch_types=scratch,
                     compiler_params=pltpu.CompilerParams(
                         use_tc_tiling_on_sc=False))


@functools.cache
def _make_sc_ea():
    scratch = [
        pltpu.VMEM((CHE, K), jnp.int32),
        pltpu.VMEM((NBUF, K, DE), jnp.float32),
        pltpu.VMEM_SHARED((NP, DE), jnp.float32),
    ] + [pltpu.SemaphoreType.DMA] * NBUF

    def body(dst_hbm, ea_hbm, ze_hbm, out_hbm,
             dst_v, eav, acc_sh, *esems):
        c = lax.axis_index("c")
        s = lax.axis_index("s")
        w = c * NS + s
        base = w * EW

        pltpu.sync_copy(dst_hbm.at[w], dst_v)
        pltpu.sync_copy(ze_hbm, acc_sh.at[pl.ds(s * SR, SR)])
        plsc.subcore_barrier()

        def ea_window(j):
            jj = jnp.minimum(j * K, EW - K)
            return ea_hbm.at[pl.ds(base + jj, K)]

        for b in range(NBUF):
            pltpu.async_copy(ea_window(b), eav.at[b], esems[b])

        @pl.loop(0, CHE, step=NBUF)
        def _(j0):
            for b in range(NBUF):
                j = j0 + b
                pltpu.make_async_copy(ea_window(j), eav.at[b],
                                      esems[b]).wait()
                pltpu.sync_copy(eav.at[b], acc_sh.at[dst_v.at[j]], add=True)

                @pl.when(j + NBUF < CHE)
                def _():
                    pltpu.async_copy(ea_window(j + NBUF), eav.at[b],
                                     esems[b])

        plsc.subcore_barrier()
        row0 = s * SR
        pltpu.sync_copy(acc_sh.at[pl.ds(row0, SR)],
                        out_hbm.at[c, pl.ds(row0, SR)])

    return pl.kernel(body,
                     out_type=jax.ShapeDtypeStruct((NC, NP, DE), jnp.float32),
                     mesh=_mesh(), scratch_types=scratch,
                     compiler_params=pltpu.CompilerParams(
                         use_tc_tiling_on_sc=False))


def _sc_round(src_sp, dst_sp, h_split, zh):
    h_cat = h_split.reshape(NC * NP, HH)
    return _make_sc_round()(src_sp, dst_sp, h_cat, zh)


def _sc_ea(dst_p, ea_p, ze):
    return _make_sc_ea()(dst_p, ea_p, ze)


def _split_out(o_ref, vals):
    o_ref[0] = vals[:, :HH]
    o_ref[1] = vals[:, HH:]


def _cat(ref):
    return jnp.concatenate([ref[0], ref[1]], axis=-1)


def _tc1_body(x_ref, wi_ref, o_ref):
    _split_out(o_ref, jnp.maximum(_dot(x_ref[...], wi_ref[...]), 0.0))


def _tc1(x_p, W_i):
    return pl.pallas_call(
        _tc1_body,
        grid=(NP // RB,),
        in_specs=[
            pl.BlockSpec((RB, D), lambda i: (i, 0)),
            pl.BlockSpec((D, H), lambda i: (0, 0)),
        ],
        out_specs=pl.BlockSpec((NC, RB, HH), lambda i: (0, i, 0)),
        out_shape=jax.ShapeDtypeStruct((NC, NP, HH), jnp.float32),
    )(x_p, W_i)


def _round_update(h0_ref, a_ref, ea_ref, wht_ref, whb_ref):
    a = _cat(a_ref)
    ea = ea_ref[0] + ea_ref[1]
    return jnp.maximum(
        _cat(h0_ref) + _dot(a, wht_ref[...]) + _dot(ea, whb_ref[...]), 0.0)


def _tc2_body(h0_ref, a_ref, ea_ref, wht_ref, whb_ref, o_ref):
    _split_out(o_ref, _round_update(h0_ref, a_ref, ea_ref, wht_ref, whb_ref))


def _tc2(h0, A, EA, wht, whb):
    return pl.pallas_call(
        _tc2_body,
        grid=(NP // RB,),
        in_specs=[
            pl.BlockSpec((NC, RB, HH), lambda i: (0, i, 0)),
            pl.BlockSpec((NC, RB, HH), lambda i: (0, i, 0)),
            pl.BlockSpec((NC, RB, DE), lambda i: (0, i, 0)),
            pl.BlockSpec((H, H), lambda i: (0, 0)),
            pl.BlockSpec((DE, H), lambda i: (0, 0)),
        ],
        out_specs=pl.BlockSpec((NC, RB, HH), lambda i: (0, i, 0)),
        out_shape=jax.ShapeDtypeStruct((NC, NP, HH), jnp.float32),
    )(h0, A, EA, wht, whb)


def _tc3_body(h0_ref, a_ref, ea_ref, x_ref, mol_ref, wht_ref, whb_ref,
              woa_ref, wob_ref, w1_ref, b1_ref, w2_ref, b2_ref,
              o_ref, acc, cnt):
    i = pl.program_id(0)

    @pl.when(i == 0)
    def _():
        acc[...] = jnp.zeros_like(acc)
        cnt[...] = jnp.zeros_like(cnt)

    h2 = _round_update(h0_ref, a_ref, ea_ref, wht_ref, whb_ref)
    atom = jnp.maximum(_dot(x_ref[...], woa_ref[...]) +
                       _dot(h2, wob_ref[...]), 0.0)
    mol_row = mol_ref[0]
    mol_ids = lax.broadcasted_iota(jnp.int32, (B, 1), 0)
    oht = (mol_ids == mol_row).astype(jnp.float32)
    acc[...] += _dot(oht, atom)
    cnt[...] += jnp.sum(oht, axis=1, keepdims=True)

    @pl.when(i == NP // RB - 1)
    def _():
        mol_vec = acc[...] / jnp.maximum(cnt[...], 1.0)
        hid = jnp.maximum(_dot(mol_vec, w1_ref[...]) + b1_ref[...], 0.0)
        o_ref[...] = _dot(hid, w2_ref[...]) + b2_ref[...]


def _tc3(h0, A, EA, x_p, mol3, wht, whb, woa, wob, W1, b1r, W2, b2r):
    return pl.pallas_call(
        _tc3_body,
        grid=(NP // RB,),
        in_specs=[
            pl.BlockSpec((NC, RB, HH), lambda i: (0, i, 0)),
            pl.BlockSpec((NC, RB, HH), lambda i: (0, i, 0)),
            pl.BlockSpec((NC, RB, DE), lambda i: (0, i, 0)),
            pl.BlockSpec((RB, D), lambda i: (i, 0)),
            pl.BlockSpec((1, 1, RB), lambda i: (i, 0, 0)),
            pl.BlockSpec((H, H), lambda i: (0, 0)),
            pl.BlockSpec((DE, H), lambda i: (0, 0)),
            pl.BlockSpec((D, H), lambda i: (0, 0)),
            pl.BlockSpec((H, H), lambda i: (0, 0)),
            pl.BlockSpec((H, FH), lambda i: (0, 0)),
            pl.BlockSpec((1, FH), lambda i: (0, 0)),
            pl.BlockSpec((FH, 1), lambda i: (0, 0)),
            pl.BlockSpec((1, 1), lambda i: (0, 0)),
        ],
        out_specs=pl.BlockSpec((B, 1), lambda i: (0, 0)),
        out_shape=jax.ShapeDtypeStruct((B, 1), jnp.float32),
        scratch_shapes=[
            pltpu.VMEM((B, H), jnp.float32),
            pltpu.VMEM((B, 1), jnp.float32),
        ],
    )(h0, A, EA, x_p, mol3, wht, whb, woa, wob, W1, b1r, W2, b2r)


def kernel(x, edge_index, edge_attr, mol_batch, W_i, W_h, W_o, W1, b1, W2,
           b2):
    f32 = jnp.float32
    npad_e = EP - E
    pad_idx = N + (jnp.arange(npad_e, dtype=jnp.int32) % (NP - N))
    src_f = jnp.concatenate([edge_index[0], pad_idx])
    dst_f = jnp.concatenate([edge_index[1], pad_idx])
    src_sp = jnp.stack(
        [src_f + c * NP for c in range(NC)]).reshape(NC, NS, CH2, K)
    dst_sp = dst_f.reshape(NS, CH2, K)
    dmat = edge_index[1].reshape(NW, EW)
    trash = N + (jnp.arange(K, dtype=jnp.int32) % (NP - N))
    full, rem = EW // K, EW % K
    parts = [dmat[:, :full * K].reshape(NW, full, K)]
    ndone = full
    if rem:
        tail = jnp.where(jnp.arange(K) >= K - rem,
                         dmat[:, EW - K:], trash[None, :])
        parts.append(tail[:, None, :])
        ndone += 1
    if CHE > ndone:
        parts.append(jnp.broadcast_to(trash, (NW, CHE - ndone, K)))
    dst_w = jnp.concatenate(parts, axis=1)
    x_p = jnp.concatenate([x, jnp.zeros((NP - N, D), f32)])
    mol3 = jnp.concatenate(
        [mol_batch, jnp.full((NP - N,), B, jnp.int32)]).reshape(
            NP // RB, 1, RB)
    zh = jnp.zeros((SR, HH), f32)
    ze = jnp.zeros((SR, DE), f32)
    wht, whb = W_h[:H], W_h[H:]
    woa, wob = W_o[:D], W_o[D:]
    b1r = b1.reshape(1, FH)
    b2r = b2.reshape(1, 1)

    h0 = _tc1(x_p, W_i)
    EA = _sc_ea(dst_w, edge_attr, ze)
    A1 = _sc_round(src_sp, dst_sp, h0, zh)
    h1 = _tc2(h0, A1, EA, wht, whb)
    A2 = _sc_round(src_sp, dst_sp, h1, zh)
    return _tc3(h0, A2, EA, x_p, mol3, wht, whb, woa, wob, W1, b1r, W2, b2r)


# speedup vs baseline: 10.3073x; 1.0296x over previous
import functools

import jax
import jax.numpy as jnp
from jax import lax
from jax.experimental import pallas as pl
from jax.experimental.pallas import tpu as pltpu
from jax.experimental.pallas import tpu_sc as plsc

N = 10000
E = 320000
D = 128
DE = 16
H = 128
FH = 128
B = 512

NC = 2
NS = 16
NW = NC * NS
K = 128
NBUF = 4
RB = 1024
NP = 10240
CH = (-(-E // (NW * K)) + NBUF - 1) // NBUF * NBUF
EP = NW * CH * K
CH2 = EP // (NS * K)
SR = NP // NS
HH = H // NC
EW = E // NW
CHE = (-(-EW // K) + NBUF - 1) // NBUF * NBUF

_PREC = lax.Precision.HIGHEST


def _dot(a, b):
    return jnp.dot(a, b, preferred_element_type=jnp.float32, precision=_PREC)


def _mesh():
    return plsc.VectorSubcoreMesh(core_axis_name="c", subcore_axis_name="s",
                                  num_cores=NC, num_subcores=NS)


@functools.cache
def _make_sc_round():
    scratch = [
        pltpu.VMEM((CH2, K), jnp.int32),
        pltpu.VMEM((CH2, K), jnp.int32),
        pltpu.VMEM((NBUF, K, HH), jnp.float32),
        pltpu.VMEM_SHARED((NP, HH), jnp.float32),
    ] + [pltpu.SemaphoreType.DMA] * NBUF

    def body(src_hbm, dst_hbm, h_hbm, zh_hbm, out_hbm,
             src_v, dst_v, rows_v, acc_sh, *gsems):
        c = lax.axis_index("c")
        s = lax.axis_index("s")

        pltpu.sync_copy(src_hbm.at[c, s], src_v)
        pltpu.sync_copy(dst_hbm.at[s], dst_v)
        pltpu.sync_copy(zh_hbm, acc_sh.at[pl.ds(s * SR, SR)])
        plsc.subcore_barrier()

        for b in range(NBUF):
            pltpu.async_copy(h_hbm.at[src_v.at[b]], rows_v.at[b], gsems[b])

        @pl.loop(0, CH2, step=NBUF)
        def _(j0):
            for b in range(NBUF):
                j = j0 + b
                pltpu.make_async_copy(
                    h_hbm.at[src_v.at[j]], rows_v.at[b], gsems[b]).wait()
                pltpu.sync_copy(rows_v.at[b], acc_sh.at[dst_v.at[j]],
                                add=True)

                @pl.when(j + NBUF < CH2)
                def _():
                    pltpu.async_copy(h_hbm.at[src_v.at[j + NBUF]],
                                     rows_v.at[b], gsems[b])

        plsc.subcore_barrier()
        row0 = s * SR
        pltpu.sync_copy(acc_sh.at[pl.ds(row0, SR)],
                        out_hbm.at[c, pl.ds(row0, SR)])

    return pl.kernel(body,
                     out_type=jax.ShapeDtypeStruct((NC, NP, HH), jnp.float32),
                     mesh=_mesh(), scratch_types=scratch,
                     compiler_params=pltpu.CompilerParams(
                         use_tc_tiling_on_sc=False))


@functools.cache
def _make_sc_ea():
    EPL = DE * K // 128
    scratch = [
        pltpu.VMEM((CHE, K), jnp.int32),
        pltpu.VMEM((NBUF, EPL, 128), jnp.float32),
        pltpu.VMEM((K, DE), jnp.float32),
        pltpu.VMEM_SHARED((NP, DE), jnp.float32),
    ] + [pltpu.SemaphoreType.DMA] * NBUF

    def body(dst_hbm, ea_hbm, ze_hbm, out_hbm,
             dst_v, eav, eap, acc_sh, *esems):
        c = lax.axis_index("c")
        s = lax.axis_index("s")
        w = c * NS + s
        base = w * EW

        pltpu.sync_copy(dst_hbm.at[w], dst_v)
        pltpu.sync_copy(ze_hbm, acc_sh.at[pl.ds(s * SR, SR)])
        plsc.subcore_barrier()

        edges_per_row = 128 // DE

        def ea_window(j):
            jj = jnp.minimum(j * K, EW - K)
            return ea_hbm.at[pl.ds((base + jj) // edges_per_row, EPL)]

        for b in range(NBUF):
            pltpu.async_copy(ea_window(b), eav.at[b], esems[b])

        @pl.loop(0, CHE, step=NBUF)
        def _(j0):
            for b in range(NBUF):
                j = j0 + b
                pltpu.make_async_copy(ea_window(j), eav.at[b],
                                      esems[b]).wait()
                for r in range(EPL):
                    for k in range(edges_per_row):
                        eap[r * edges_per_row + k, :] = (
                            eav[b, r, pl.ds(DE * k, DE)])
                pltpu.sync_copy(eap, acc_sh.at[dst_v.at[j]], add=True)

                @pl.when(j + NBUF < CHE)
                def _():
                    pltpu.async_copy(ea_window(j + NBUF), eav.at[b],
                                     esems[b])

        plsc.subcore_barrier()
        row0 = s * SR
        pltpu.sync_copy(acc_sh.at[pl.ds(row0, SR)],
                        out_hbm.at[c, pl.ds(row0, SR)])

    return pl.kernel(body,
                     out_type=jax.ShapeDtypeStruct((NC, NP, DE), jnp.float32),
                     mesh=_mesh(), scratch_types=scratch,
                     compiler_params=pltpu.CompilerParams(
                         use_tc_tiling_on_sc=False))


def _sc_round(src_sp, dst_sp, h_split, zh):
    h_cat = h_split.reshape(NC * NP, HH)
    return _make_sc_round()(src_sp, dst_sp, h_cat, zh)


def _sc_ea(dst_p, ea_p, ze):
    return _make_sc_ea()(dst_p, ea_p, ze)


def _split_out(o_ref, vals):
    o_ref[0] = vals[:, :HH]
    o_ref[1] = vals[:, HH:]


def _cat(ref):
    return jnp.concatenate([ref[0], ref[1]], axis=-1)


def _tc1_body(x_ref, wi_ref, o_ref):
    _split_out(o_ref, jnp.maximum(_dot(x_ref[...], wi_ref[...]), 0.0))


def _tc1(x_p, W_i):
    return pl.pallas_call(
        _tc1_body,
        grid=(NP // RB,),
        in_specs=[
            pl.BlockSpec((RB, D), lambda i: (i, 0)),
            pl.BlockSpec((D, H), lambda i: (0, 0)),
        ],
        out_specs=pl.BlockSpec((NC, RB, HH), lambda i: (0, i, 0)),
        out_shape=jax.ShapeDtypeStruct((NC, NP, HH), jnp.float32),
    )(x_p, W_i)


def _round_update(h0_ref, a_ref, ea_ref, wht_ref, whb_ref):
    a = _cat(a_ref)
    ea = ea_ref[0] + ea_ref[1]
    return jnp.maximum(
        _cat(h0_ref) + _dot(a, wht_ref[...]) + _dot(ea, whb_ref[...]), 0.0)


def _tc2_body(h0_ref, a_ref, ea_ref, wht_ref, whb_ref, o_ref):
    _split_out(o_ref, _round_update(h0_ref, a_ref, ea_ref, wht_ref, whb_ref))


def _tc2(h0, A, EA, wht, whb):
    return pl.pallas_call(
        _tc2_body,
        grid=(NP // RB,),
        in_specs=[
            pl.BlockSpec((NC, RB, HH), lambda i: (0, i, 0)),
            pl.BlockSpec((NC, RB, HH), lambda i: (0, i, 0)),
            pl.BlockSpec((NC, RB, DE), lambda i: (0, i, 0)),
            pl.BlockSpec((H, H), lambda i: (0, 0)),
            pl.BlockSpec((DE, H), lambda i: (0, 0)),
        ],
        out_specs=pl.BlockSpec((NC, RB, HH), lambda i: (0, i, 0)),
        out_shape=jax.ShapeDtypeStruct((NC, NP, HH), jnp.float32),
    )(h0, A, EA, wht, whb)


def _tc3_body(h0_ref, a_ref, ea_ref, x_ref, mol_ref, wht_ref, whb_ref,
              woa_ref, wob_ref, w1_ref, b1_ref, w2_ref, b2_ref,
              o_ref, acc, cnt):
    i = pl.program_id(0)

    @pl.when(i == 0)
    def _():
        acc[...] = jnp.zeros_like(acc)
        cnt[...] = jnp.zeros_like(cnt)

    h2 = _round_update(h0_ref, a_ref, ea_ref, wht_ref, whb_ref)
    atom = jnp.maximum(_dot(x_ref[...], woa_ref[...]) +
                       _dot(h2, wob_ref[...]), 0.0)
    mol_row = mol_ref[0]
    mol_ids = lax.broadcasted_iota(jnp.int32, (B, 1), 0)
    oht = (mol_ids == mol_row).astype(jnp.float32)
    acc[...] += _dot(oht, atom)
    cnt[...] += jnp.sum(oht, axis=1, keepdims=True)

    @pl.when(i == NP // RB - 1)
    def _():
        mol_vec = acc[...] / jnp.maximum(cnt[...], 1.0)
        hid = jnp.maximum(_dot(mol_vec, w1_ref[...]) + b1_ref[...], 0.0)
        o_ref[...] = _dot(hid, w2_ref[...]) + b2_ref[...]


def _tc3(h0, A, EA, x_p, mol3, wht, whb, woa, wob, W1, b1r, W2, b2r):
    return pl.pallas_call(
        _tc3_body,
        grid=(NP // RB,),
        in_specs=[
            pl.BlockSpec((NC, RB, HH), lambda i: (0, i, 0)),
            pl.BlockSpec((NC, RB, HH), lambda i: (0, i, 0)),
            pl.BlockSpec((NC, RB, DE), lambda i: (0, i, 0)),
            pl.BlockSpec((RB, D), lambda i: (i, 0)),
            pl.BlockSpec((1, 1, RB), lambda i: (i, 0, 0)),
            pl.BlockSpec((H, H), lambda i: (0, 0)),
            pl.BlockSpec((DE, H), lambda i: (0, 0)),
            pl.BlockSpec((D, H), lambda i: (0, 0)),
            pl.BlockSpec((H, H), lambda i: (0, 0)),
            pl.BlockSpec((H, FH), lambda i: (0, 0)),
            pl.BlockSpec((1, FH), lambda i: (0, 0)),
            pl.BlockSpec((FH, 1), lambda i: (0, 0)),
            pl.BlockSpec((1, 1), lambda i: (0, 0)),
        ],
        out_specs=pl.BlockSpec((B, 1), lambda i: (0, 0)),
        out_shape=jax.ShapeDtypeStruct((B, 1), jnp.float32),
        scratch_shapes=[
            pltpu.VMEM((B, H), jnp.float32),
            pltpu.VMEM((B, 1), jnp.float32),
        ],
    )(h0, A, EA, x_p, mol3, wht, whb, woa, wob, W1, b1r, W2, b2r)


def kernel(x, edge_index, edge_attr, mol_batch, W_i, W_h, W_o, W1, b1, W2,
           b2):
    f32 = jnp.float32
    npad_e = EP - E
    pad_idx = N + (jnp.arange(npad_e, dtype=jnp.int32) % (NP - N))
    src_f = jnp.concatenate([edge_index[0], pad_idx])
    dst_f = jnp.concatenate([edge_index[1], pad_idx])
    src_sp = jnp.stack(
        [src_f + c * NP for c in range(NC)]).reshape(NC, NS, CH2, K)
    dst_sp = dst_f.reshape(NS, CH2, K)
    dmat = edge_index[1].reshape(NW, EW)
    trash = N + (jnp.arange(K, dtype=jnp.int32) % (NP - N))
    full, rem = EW // K, EW % K
    parts = [dmat[:, :full * K].reshape(NW, full, K)]
    ndone = full
    if rem:
        tail = jnp.where(jnp.arange(K) >= K - rem,
                         dmat[:, EW - K:], trash[None, :])
        parts.append(tail[:, None, :])
        ndone += 1
    if CHE > ndone:
        parts.append(jnp.broadcast_to(trash, (NW, CHE - ndone, K)))
    dst_w = jnp.concatenate(parts, axis=1)
    x_p = jnp.concatenate([x, jnp.zeros((NP - N, D), f32)])
    mol3 = jnp.concatenate(
        [mol_batch, jnp.full((NP - N,), B, jnp.int32)]).reshape(
            NP // RB, 1, RB)
    zh = jnp.zeros((SR, HH), f32)
    ze = jnp.zeros((SR, DE), f32)
    wht, whb = W_h[:H], W_h[H:]
    woa, wob = W_o[:D], W_o[D:]
    b1r = b1.reshape(1, FH)
    b2r = b2.reshape(1, 1)

    h0 = _tc1(x_p, W_i)
    EA = _sc_ea(dst_w, edge_attr.reshape(E * DE // 128, 128), ze)
    A1 = _sc_round(src_sp, dst_sp, h0, zh)
    h1 = _tc2(h0, A1, EA, wht, whb)
    A2 = _sc_round(src_sp, dst_sp, h1, zh)
    return _tc3(h0, A2, EA, x_p, mol3, wht, whb, woa, wob, W1, b1r, W2, b2r)
